# Initial kernel scaffold; baseline (speedup 1.0000x reference)
import jax
import jax.numpy as jnp
from jax.experimental import pallas as pl


def kernel(X, F, N, Q, W):
    raise NotImplementedError("write your pallas kernel here")


# speedup vs baseline: 14.1081x; 14.1081x over previous
import functools

import jax
import jax.numpy as jnp
from jax import lax
from jax.experimental import pallas as pl
from jax.experimental.pallas import tpu as pltpu
from jax.experimental.pallas import tpu_sc as plsc

NC, NS, L = 2, 16, 16
NW = NC * NS

NPTS = 10000
K = 32
DIN = 128
NPAD = 10240
QPW = NPAD // NW
E = NPAD * K
EPW = E // NW
GQ = 4
GK = GQ * K
NB = QPW // GQ


def _sc_coord_deltas(xx, xy, xz, nidx):
    mesh = plsc.VectorSubcoreMesh(core_axis_name="c", subcore_axis_name="s")
    f32 = jnp.float32

    @functools.partial(
        pl.kernel,
        out_type=(
            jax.ShapeDtypeStruct((E,), f32),
            jax.ShapeDtypeStruct((E,), f32),
            jax.ShapeDtypeStruct((E,), f32),
        ),
        mesh=mesh,
        compiler_params=pltpu.CompilerParams(needs_layout_passes=False),
        scratch_types=[
            pltpu.VMEM((NPAD,), f32),
            pltpu.VMEM((NPAD,), f32),
            pltpu.VMEM((NPAD,), f32),
            pltpu.VMEM((EPW,), jnp.int32),
            pltpu.VMEM((EPW,), f32),
            pltpu.VMEM((EPW,), f32),
            pltpu.VMEM((EPW,), f32),
        ],
    )
    def body(xx_hbm, xy_hbm, xz_hbm, nidx_hbm, dx_hbm, dy_hbm, dz_hbm,
             xx_v, xy_v, xz_v, idx_v, dx_v, dy_v, dz_v):
        wid = lax.axis_index("s") * NC + lax.axis_index("c")
        base = wid * EPW
        pltpu.sync_copy(xx_hbm, xx_v)
        pltpu.sync_copy(xy_hbm, xy_v)
        pltpu.sync_copy(xz_hbm, xz_v)
        pltpu.sync_copy(nidx_hbm.at[pl.ds(base, EPW)], idx_v)

        def chunk(i, _):
            sl = pl.ds(i * L, L)
            nbr = idx_v[sl]
            eid = base + i * L + lax.iota(jnp.int32, L)
            q = lax.shift_right_logical(eid, 5)
            dx_v[sl] = plsc.load_gather(xx_v, [nbr]) - plsc.load_gather(xx_v, [q])
            dy_v[sl] = plsc.load_gather(xy_v, [nbr]) - plsc.load_gather(xy_v, [q])
            dz_v[sl] = plsc.load_gather(xz_v, [nbr]) - plsc.load_gather(xz_v, [q])
            return 0

        lax.fori_loop(0, EPW // L, chunk, 0)
        pltpu.sync_copy(dx_v, dx_hbm.at[pl.ds(base, EPW)])
        pltpu.sync_copy(dy_v, dy_hbm.at[pl.ds(base, EPW)])
        pltpu.sync_copy(dz_v, dz_hbm.at[pl.ds(base, EPW)])

    return body(xx, xy, xz, nidx)


def _tc_weights_body(qx_ref, qy_ref, qz_ref, dx_ref, dy_ref, dz_ref, w_ref):
    dx = dx_ref[:, :]
    dy = dy_ref[:, :]
    dz = dz_ref[:, :]
    acc = jnp.zeros_like(dx)
    for p in range(19):
        ax = dx - qx_ref[p]
        ay = dy - qy_ref[p]
        az = dz - qz_ref[p]
        d = jnp.sqrt(ax * ax + ay * ay + az * az)
        acc = acc + jnp.maximum(0.0, 1.0 - d)
    w_ref[:, :] = acc


def _tc_weights(qx, qy, qz, dx2, dy2, dz2):
    rows = E // 128
    blk = rows // 8
    grid = (8,)
    smem = pl.BlockSpec(memory_space=pltpu.SMEM)
    vspec = pl.BlockSpec((blk, 128), lambda i: (i, 0))
    return pl.pallas_call(
        _tc_weights_body,
        grid=grid,
        in_specs=[smem, smem, smem, vspec, vspec, vspec],
        out_specs=vspec,
        out_shape=jax.ShapeDtypeStruct((rows, 128), jnp.float32),
    )(qx, qy, qz, dx2, dy2, dz2)


def _sc_accumulate(f, nidx, w):
    mesh = plsc.VectorSubcoreMesh(core_axis_name="c", subcore_axis_name="s")
    f32 = jnp.float32

    @functools.partial(
        pl.kernel,
        out_type=jax.ShapeDtypeStruct((NPAD, DIN), f32),
        mesh=mesh,
        compiler_params=pltpu.CompilerParams(needs_layout_passes=False),
        scratch_types=[
            pltpu.VMEM((EPW,), jnp.int32),
            pltpu.VMEM((EPW + L,), f32),
            pltpu.VMEM((2, GK, DIN), f32),
            pltpu.VMEM((QPW, DIN), f32),
            pltpu.SemaphoreType.DMA,
            pltpu.SemaphoreType.DMA,
        ],
    )
    def body(f_hbm, nidx_hbm, w_hbm, s_hbm, idx_v, w_v, rows_v, s_v, sem0, sem1):
        wid = lax.axis_index("s") * NC + lax.axis_index("c")
        ebase = wid * EPW
        qbase = wid * QPW
        sems = (sem0, sem1)
        pltpu.sync_copy(nidx_hbm.at[pl.ds(ebase, EPW)], idx_v)
        pltpu.sync_copy(w_hbm.at[pl.ds(ebase, EPW)], w_v.at[pl.ds(0, EPW)])

        def start(b, buf):
            pltpu.async_copy(
                f_hbm.at[idx_v.at[pl.ds(b * GK, GK)]], rows_v.at[buf], sems[buf])

        start(0, 0)
        start(1, 1)

        def outer(ob, _):
            for par in range(2):
                b = ob * 2 + par
                pltpu.make_async_copy(
                    f_hbm.at[idx_v.at[pl.ds(b * GK, GK)]],
                    rows_v.at[par], sems[par]).wait()

                for g in range(GQ):
                    def edge(j, acc):
                        e = b * GK + g * K + j
                        ws = w_v[pl.ds(e, L)][0]
                        row = g * K + j
                        return tuple(
                            acc[c] + ws * rows_v[par, row, pl.ds(c * L, L)]
                            for c in range(DIN // L))

                    acc = lax.fori_loop(
                        0, K, edge,
                        tuple(jnp.zeros((L,), f32) for _ in range(DIN // L)))
                    for c in range(DIN // L):
                        s_v[b * GQ + g, pl.ds(c * L, L)] = acc[c]

                nb = b + 2

                @pl.when(nb < NB)
                def _():
                    start(nb, par)
            return 0

        lax.fori_loop(0, NB // 2, outer, 0)
        pltpu.sync_copy(s_v, s_hbm.at[pl.ds(qbase, QPW)])

    return body(f, nidx, w)


def _tc_matmul_body(s_ref, w_ref, o_ref):
    o_ref[:, :] = jnp.dot(s_ref[:, :], w_ref[:, :],
                          preferred_element_type=jnp.float32)


def _tc_matmul(s, w):
    blk = NPAD // 8
    return pl.pallas_call(
        _tc_matmul_body,
        grid=(8,),
        in_specs=[
            pl.BlockSpec((blk, DIN), lambda i: (i, 0)),
            pl.BlockSpec((DIN, DIN), lambda i: (0, 0)),
        ],
        out_specs=pl.BlockSpec((blk, DIN), lambda i: (i, 0)),
        out_shape=jax.ShapeDtypeStruct((NPAD, DIN), jnp.float32),
    )(s, w)


def kernel(X, F, N, Q, W):
    B = X.shape[0]
    x = X[0]
    f = F[0]
    n = N[0]

    xp = jnp.pad(x, ((0, NPAD - NPTS), (0, 0)))
    xx, xy, xz = xp[:, 0], xp[:, 1], xp[:, 2]
    nidx = jnp.pad(n, ((0, NPAD - NPTS), (0, 0))).reshape(E)

    dx, dy, dz = _sc_coord_deltas(xx, xy, xz, nidx)

    qx, qy, qz = Q[:, 0], Q[:, 1], Q[:, 2]
    w_edges = _tc_weights(qx, qy, qz,
                          dx.reshape(E // 128, 128),
                          dy.reshape(E // 128, 128),
                          dz.reshape(E // 128, 128))

    s = _sc_accumulate(f, nidx, w_edges.reshape(E))
    out = _tc_matmul(s, W)
    return out[:NPTS][None]


# speedup vs baseline: 14.2562x; 1.0105x over previous
import functools

import jax
import jax.numpy as jnp
from jax import lax
from jax.experimental import pallas as pl
from jax.experimental.pallas import tpu as pltpu
from jax.experimental.pallas import tpu_sc as plsc

NC, NS, L = 2, 16, 16
NW = NC * NS

NPTS = 10000
K = 32
DIN = 128
NPAD = 10240
QPW = NPAD // NW
E = NPAD * K
EPW = E // NW
GQ = 4
GK = GQ * K
NB = QPW // GQ


def _sc_coord_deltas(xx, xy, xz, nidx):
    mesh = plsc.VectorSubcoreMesh(core_axis_name="c", subcore_axis_name="s")
    f32 = jnp.float32

    @functools.partial(
        pl.kernel,
        out_type=(
            jax.ShapeDtypeStruct((E,), f32),
            jax.ShapeDtypeStruct((E,), f32),
            jax.ShapeDtypeStruct((E,), f32),
        ),
        mesh=mesh,
        compiler_params=pltpu.CompilerParams(needs_layout_passes=False),
        scratch_types=[
            pltpu.VMEM((NPAD,), f32),
            pltpu.VMEM((NPAD,), f32),
            pltpu.VMEM((NPAD,), f32),
            pltpu.VMEM((EPW,), jnp.int32),
            pltpu.VMEM((EPW,), f32),
            pltpu.VMEM((EPW,), f32),
            pltpu.VMEM((EPW,), f32),
        ],
    )
    def body(xx_hbm, xy_hbm, xz_hbm, nidx_hbm, dx_hbm, dy_hbm, dz_hbm,
             xx_v, xy_v, xz_v, idx_v, dx_v, dy_v, dz_v):
        wid = lax.axis_index("s") * NC + lax.axis_index("c")
        base = wid * EPW
        pltpu.sync_copy(xx_hbm, xx_v)
        pltpu.sync_copy(xy_hbm, xy_v)
        pltpu.sync_copy(xz_hbm, xz_v)
        pltpu.sync_copy(nidx_hbm.at[pl.ds(base, EPW)], idx_v)

        def chunk(i, _):
            sl = pl.ds(i * L, L)
            nbr = idx_v[sl]
            eid = base + i * L + lax.iota(jnp.int32, L)
            q = lax.shift_right_logical(eid, 5)
            dx_v[sl] = plsc.load_gather(xx_v, [nbr]) - plsc.load_gather(xx_v, [q])
            dy_v[sl] = plsc.load_gather(xy_v, [nbr]) - plsc.load_gather(xy_v, [q])
            dz_v[sl] = plsc.load_gather(xz_v, [nbr]) - plsc.load_gather(xz_v, [q])
            return 0

        lax.fori_loop(0, EPW // L, chunk, 0)
        pltpu.sync_copy(dx_v, dx_hbm.at[pl.ds(base, EPW)])
        pltpu.sync_copy(dy_v, dy_hbm.at[pl.ds(base, EPW)])
        pltpu.sync_copy(dz_v, dz_hbm.at[pl.ds(base, EPW)])

    return body(xx, xy, xz, nidx)


def _tc_weights_body(qx_ref, qy_ref, qz_ref, dx_ref, dy_ref, dz_ref, w_ref):
    dx = dx_ref[:, :]
    dy = dy_ref[:, :]
    dz = dz_ref[:, :]
    acc = jnp.zeros_like(dx)
    for p in range(19):
        ax = dx - qx_ref[p]
        ay = dy - qy_ref[p]
        az = dz - qz_ref[p]
        d = jnp.sqrt(ax * ax + ay * ay + az * az)
        acc = acc + jnp.maximum(0.0, 1.0 - d)
    w_ref[:, :] = acc


def _tc_weights(qx, qy, qz, dx2, dy2, dz2):
    rows = E // 128
    blk = rows // 8
    grid = (8,)
    smem = pl.BlockSpec(memory_space=pltpu.SMEM)
    vspec = pl.BlockSpec((blk, 128), lambda i: (i, 0))
    return pl.pallas_call(
        _tc_weights_body,
        grid=grid,
        in_specs=[smem, smem, smem, vspec, vspec, vspec],
        out_specs=vspec,
        out_shape=jax.ShapeDtypeStruct((rows, 128), jnp.float32),
    )(qx, qy, qz, dx2, dy2, dz2)


def _sc_accumulate(f, nidx, w):
    mesh = plsc.VectorSubcoreMesh(core_axis_name="c", subcore_axis_name="s")
    f32 = jnp.float32

    @functools.partial(
        pl.kernel,
        out_type=jax.ShapeDtypeStruct((NPAD, DIN), f32),
        mesh=mesh,
        compiler_params=pltpu.CompilerParams(needs_layout_passes=False),
        scratch_types=[
            pltpu.VMEM((EPW,), jnp.int32),
            pltpu.VMEM((EPW + L,), f32),
            pltpu.VMEM((2, GK, DIN), f32),
            pltpu.VMEM((QPW, DIN), f32),
            pltpu.SemaphoreType.DMA,
            pltpu.SemaphoreType.DMA,
        ],
    )
    def body(f_hbm, nidx_hbm, w_hbm, s_hbm, idx_v, w_v, rows_v, s_v, sem0, sem1):
        wid = lax.axis_index("s") * NC + lax.axis_index("c")
        ebase = wid * EPW
        qbase = wid * QPW
        sems = (sem0, sem1)
        pltpu.sync_copy(nidx_hbm.at[pl.ds(ebase, EPW)], idx_v)
        pltpu.sync_copy(w_hbm.at[pl.ds(ebase, EPW)], w_v.at[pl.ds(0, EPW)])

        def start(b, buf):
            pltpu.async_copy(
                f_hbm.at[idx_v.at[pl.ds(b * GK, GK)]], rows_v.at[buf], sems[buf])

        start(0, 0)
        start(1, 1)

        def outer(ob, _):
            for par in range(2):
                b = ob * 2 + par
                pltpu.make_async_copy(
                    f_hbm.at[idx_v.at[pl.ds(b * GK, GK)]],
                    rows_v.at[par], sems[par]).wait()

                for g in range(GQ):
                    wq = [w_v[pl.ds(b * GK + g * K + h * L, L)]
                          for h in range(K // L)]
                    acc = [jnp.zeros((L,), f32) for _ in range(DIN // L)]
                    for j in range(K):
                        ws = wq[j // L][j % L]
                        row = g * K + j
                        for c in range(DIN // L):
                            acc[c] = acc[c] + ws * rows_v[par, row,
                                                          pl.ds(c * L, L)]
                    for c in range(DIN // L):
                        s_v[b * GQ + g, pl.ds(c * L, L)] = acc[c]

                nb = b + 2

                @pl.when(nb < NB)
                def _():
                    start(nb, par)
            return 0

        lax.fori_loop(0, NB // 2, outer, 0)
        pltpu.sync_copy(s_v, s_hbm.at[pl.ds(qbase, QPW)])

    return body(f, nidx, w)


def _tc_matmul_body(s_ref, w_ref, o_ref):
    o_ref[:, :] = jnp.dot(s_ref[:, :], w_ref[:, :],
                          preferred_element_type=jnp.float32)


def _tc_matmul(s, w):
    blk = NPAD // 8
    return pl.pallas_call(
        _tc_matmul_body,
        grid=(8,),
        in_specs=[
            pl.BlockSpec((blk, DIN), lambda i: (i, 0)),
            pl.BlockSpec((DIN, DIN), lambda i: (0, 0)),
        ],
        out_specs=pl.BlockSpec((blk, DIN), lambda i: (i, 0)),
        out_shape=jax.ShapeDtypeStruct((NPAD, DIN), jnp.float32),
    )(s, w)


def kernel(X, F, N, Q, W):
    B = X.shape[0]
    x = X[0]
    f = F[0]
    n = N[0]

    xp = jnp.pad(x, ((0, NPAD - NPTS), (0, 0)))
    xx, xy, xz = xp[:, 0], xp[:, 1], xp[:, 2]
    nidx = jnp.pad(n, ((0, NPAD - NPTS), (0, 0))).reshape(E)

    dx, dy, dz = _sc_coord_deltas(xx, xy, xz, nidx)

    qx, qy, qz = Q[:, 0], Q[:, 1], Q[:, 2]
    w_edges = _tc_weights(qx, qy, qz,
                          dx.reshape(E // 128, 128),
                          dy.reshape(E // 128, 128),
                          dz.reshape(E // 128, 128))

    s = _sc_accumulate(f, nidx, w_edges.reshape(E))
    out = _tc_matmul(s, W)
    return out[:NPTS][None]


# speedup vs baseline: 25.0418x; 1.7566x over previous
import functools

import jax
import jax.numpy as jnp
from jax import lax
from jax.experimental import pallas as pl
from jax.experimental.pallas import tpu as pltpu
from jax.experimental.pallas import tpu_sc as plsc

NC, NS, L = 2, 16, 16
NW = NC * NS

NPTS = 10000
K = 32
DIN = 128
NPAD = 10240
QPW = NPAD // NW
E = NPAD * K
EPW = E // NW
GQ = 4
GK = GQ * K
NB = QPW // GQ


def _sc_coord_deltas(xx, xy, xz, nidx):
    mesh = plsc.VectorSubcoreMesh(core_axis_name="c", subcore_axis_name="s")
    f32 = jnp.float32

    @functools.partial(
        pl.kernel,
        out_type=(
            jax.ShapeDtypeStruct((E,), f32),
            jax.ShapeDtypeStruct((E,), f32),
            jax.ShapeDtypeStruct((E,), f32),
        ),
        mesh=mesh,
        compiler_params=pltpu.CompilerParams(needs_layout_passes=False),
        scratch_types=[
            pltpu.VMEM((NPAD,), f32),
            pltpu.VMEM((NPAD,), f32),
            pltpu.VMEM((NPAD,), f32),
            pltpu.VMEM((EPW,), jnp.int32),
            pltpu.VMEM((EPW,), f32),
            pltpu.VMEM((EPW,), f32),
            pltpu.VMEM((EPW,), f32),
        ],
    )
    def body(xx_hbm, xy_hbm, xz_hbm, nidx_hbm, dx_hbm, dy_hbm, dz_hbm,
             xx_v, xy_v, xz_v, idx_v, dx_v, dy_v, dz_v):
        wid = lax.axis_index("s") * NC + lax.axis_index("c")
        base = wid * EPW
        pltpu.sync_copy(xx_hbm, xx_v)
        pltpu.sync_copy(xy_hbm, xy_v)
        pltpu.sync_copy(xz_hbm, xz_v)
        pltpu.sync_copy(nidx_hbm.at[pl.ds(base, EPW)], idx_v)

        def chunk(i, _):
            sl = pl.ds(i * L, L)
            nbr = idx_v[sl]
            eid = base + i * L + lax.iota(jnp.int32, L)
            q = lax.shift_right_logical(eid, 5)
            dx_v[sl] = plsc.load_gather(xx_v, [nbr]) - plsc.load_gather(xx_v, [q])
            dy_v[sl] = plsc.load_gather(xy_v, [nbr]) - plsc.load_gather(xy_v, [q])
            dz_v[sl] = plsc.load_gather(xz_v, [nbr]) - plsc.load_gather(xz_v, [q])
            return 0

        lax.fori_loop(0, EPW // L, chunk, 0)
        pltpu.sync_copy(dx_v, dx_hbm.at[pl.ds(base, EPW)])
        pltpu.sync_copy(dy_v, dy_hbm.at[pl.ds(base, EPW)])
        pltpu.sync_copy(dz_v, dz_hbm.at[pl.ds(base, EPW)])

    return body(xx, xy, xz, nidx)


def _tc_weights_body(qx_ref, qy_ref, qz_ref, dx_ref, dy_ref, dz_ref, w_ref):
    dx = dx_ref[:, :]
    dy = dy_ref[:, :]
    dz = dz_ref[:, :]
    acc = jnp.zeros_like(dx)
    for p in range(19):
        ax = dx - qx_ref[p]
        ay = dy - qy_ref[p]
        az = dz - qz_ref[p]
        d = jnp.sqrt(ax * ax + ay * ay + az * az)
        acc = acc + jnp.maximum(0.0, 1.0 - d)
    w_ref[:, :] = acc


def _tc_weights(qx, qy, qz, dx2, dy2, dz2):
    rows = E // 128
    blk = rows // 8
    grid = (8,)
    smem = pl.BlockSpec(memory_space=pltpu.SMEM)
    vspec = pl.BlockSpec((blk, 128), lambda i: (i, 0))
    return pl.pallas_call(
        _tc_weights_body,
        grid=grid,
        in_specs=[smem, smem, smem, vspec, vspec, vspec],
        out_specs=vspec,
        out_shape=jax.ShapeDtypeStruct((rows, 128), jnp.float32),
    )(qx, qy, qz, dx2, dy2, dz2)


def _sc_accumulate(f, nidx, w):
    mesh = plsc.VectorSubcoreMesh(core_axis_name="c", subcore_axis_name="s")
    f32 = jnp.float32

    @functools.partial(
        pl.kernel,
        out_type=jax.ShapeDtypeStruct((NPAD, DIN), f32),
        mesh=mesh,
        compiler_params=pltpu.CompilerParams(
            needs_layout_passes=False, use_tc_tiling_on_sc=False),
        scratch_types=[
            pltpu.VMEM((EPW,), jnp.int32),
            pltpu.VMEM((EPW + L,), f32),
            pltpu.VMEM((2, GK, DIN // 2), jnp.int32),
            pltpu.VMEM((QPW, DIN), f32),
            pltpu.SemaphoreType.DMA,
            pltpu.SemaphoreType.DMA,
        ],
    )
    def body(f_hbm, nidx_hbm, w_hbm, s_hbm, idx_v, w_v, rows_v, s_v, sem0, sem1):
        wid = lax.axis_index("s") * NC + lax.axis_index("c")
        ebase = wid * EPW
        qbase = wid * QPW
        sems = (sem0, sem1)
        pltpu.sync_copy(nidx_hbm.at[pl.ds(ebase, EPW)], idx_v)
        pltpu.sync_copy(w_hbm.at[pl.ds(ebase, EPW)], w_v.at[pl.ds(0, EPW)])

        def start(b, buf):
            pltpu.async_copy(
                f_hbm.at[idx_v.at[pl.ds(b * GK, GK)]], rows_v.at[buf], sems[buf])

        start(0, 0)
        start(1, 1)

        def outer(ob, _):
            for par in range(2):
                b = ob * 2 + par
                pltpu.make_async_copy(
                    f_hbm.at[idx_v.at[pl.ds(b * GK, GK)]],
                    rows_v.at[par], sems[par]).wait()

                for g in range(GQ):
                    wq = [w_v[pl.ds(b * GK + g * K + h * L, L)]
                          for h in range(K // L)]
                    acc = [jnp.zeros((L,), f32) for _ in range(DIN // L)]
                    for j in range(K):
                        ws = wq[j // L][j % L]
                        row = g * K + j
                        for c in range(DIN // (2 * L)):
                            v = plsc.bitcast(
                                rows_v[par, row, pl.ds(c * L, L)],
                                jnp.bfloat16)
                            ev, od = plsc.unpack(
                                v, format=plsc.PackFormat.INTERLEAVED)
                            acc[2 * c] = acc[2 * c] + ws * ev
                            acc[2 * c + 1] = acc[2 * c + 1] + ws * od
                    for c in range(DIN // L):
                        s_v[b * GQ + g, pl.ds(c * L, L)] = acc[c]

                nb = b + 2

                @pl.when(nb < NB)
                def _():
                    start(nb, par)
            return 0

        lax.fori_loop(0, NB // 2, outer, 0)
        pltpu.sync_copy(s_v, s_hbm.at[pl.ds(qbase, QPW)])

    return body(f, nidx, w)


def _tc_matmul_body(s_ref, w_ref, o_ref):
    o_ref[:, :] = jnp.dot(s_ref[:, :], w_ref[:, :],
                          preferred_element_type=jnp.float32)


def _tc_matmul(s, w):
    blk = NPAD // 8
    return pl.pallas_call(
        _tc_matmul_body,
        grid=(8,),
        in_specs=[
            pl.BlockSpec((blk, DIN), lambda i: (i, 0)),
            pl.BlockSpec((DIN, DIN), lambda i: (0, 0)),
        ],
        out_specs=pl.BlockSpec((blk, DIN), lambda i: (i, 0)),
        out_shape=jax.ShapeDtypeStruct((NPAD, DIN), jnp.float32),
    )(s, w)


_PERM = sum(([c * 32 + i for i in range(0, 32, 2)]
             + [c * 32 + i for i in range(1, 32, 2)]
             for c in range(DIN // 32)), [])


def kernel(X, F, N, Q, W):
    B = X.shape[0]
    x = X[0]
    fb = F[0].astype(jnp.bfloat16).reshape(NPTS, DIN // 2, 2)
    f = lax.bitcast_convert_type(fb, jnp.int32)
    n = N[0]

    xp = jnp.pad(x, ((0, NPAD - NPTS), (0, 0)))
    xx, xy, xz = xp[:, 0], xp[:, 1], xp[:, 2]
    nidx = jnp.pad(n, ((0, NPAD - NPTS), (0, 0))).reshape(E)

    dx, dy, dz = _sc_coord_deltas(xx, xy, xz, nidx)

    qx, qy, qz = Q[:, 0], Q[:, 1], Q[:, 2]
    w_edges = _tc_weights(qx, qy, qz,
                          dx.reshape(E // 128, 128),
                          dy.reshape(E // 128, 128),
                          dz.reshape(E // 128, 128))

    s = _sc_accumulate(f, nidx, w_edges.reshape(E))
    out = _tc_matmul(s, W[jnp.array(_PERM, dtype=jnp.int32)])
    return out[:NPTS][None]
